# baseline (speedup 1.0000x reference)
import functools

import jax
import jax.numpy as jnp
from jax import lax
from jax.experimental import pallas as pl
from jax.experimental.pallas import tpu as pltpu
from jax.experimental.pallas import tpu_sc as plsc

DIM = 1024
HALF = 512
L0 = 64
L1 = 64
TOTAL = L0 * L1
NW = 32
ROWS_PER_W = TOTAL // NW
LANES = 16
NCHUNK = HALF // LANES

_TWO_PI = 6.283185307179586
_PI = 3.141592653589793
_HALF_PI = 1.5707963267948966


def _sin_vec(x):
    y = x * (1.0 / _TWO_PI)
    k = jnp.where(y >= 0, y + 0.5, y - 0.5).astype(jnp.int32).astype(jnp.float32)
    r = x - k * _TWO_PI
    r = jnp.where(r > _HALF_PI, _PI - r, r)
    r = jnp.where(r < -_HALF_PI, -_PI - r, r)
    r2 = r * r
    p = jnp.float32(-2.5052108385441718e-08)
    p = p * r2 + 2.7557319223985893e-06
    p = p * r2 - 0.0001984126984126984
    p = p * r2 + 0.008333333333333333
    p = p * r2 - 0.16666666666666666
    p = p * r2 + 1.0
    return r * p


def _lane_broadcast(v, lane):
    idx = jnp.full((LANES, 1), lane, dtype=jnp.int32)
    dnums = lax.GatherDimensionNumbers(
        offset_dims=(), collapsed_slice_dims=(0,), start_index_map=(0,))
    return lax.gather(v, idx, dnums, (1,),
                      mode=lax.GatherScatterMode.PROMISE_IN_BOUNDS)


def _make_sc_kernel():
    mesh = plsc.VectorSubcoreMesh(core_axis_name="c", subcore_axis_name="s")

    @functools.partial(
        pl.kernel,
        mesh=mesh,
        out_type=jax.ShapeDtypeStruct((TOTAL, DIM), jnp.float32),
        scratch_types=[
            pltpu.VMEM((HALF,), jnp.float32),
            pltpu.VMEM((HALF,), jnp.float32),
            pltpu.VMEM((HALF,), jnp.float32),
            pltpu.VMEM((HALF,), jnp.float32),
            pltpu.VMEM((2 * LANES,), jnp.float32),
            pltpu.VMEM((HALF,), jnp.float32),
            pltpu.VMEM((HALF,), jnp.float32),
            pltpu.VMEM((HALF,), jnp.float32),
            pltpu.VMEM((HALF,), jnp.float32),
            pltpu.VMEM((HALF,), jnp.float32),
            pltpu.VMEM((HALF,), jnp.float32),
            pltpu.VMEM((4, HALF), jnp.float32),
            pltpu.VMEM((4, HALF), jnp.float32),
            pltpu.VMEM((L1, HALF), jnp.float32),
            pltpu.SemaphoreType.DMA,
            pltpu.SemaphoreType.DMA,
        ],
    )
    def sc_kernel(w0_h, b0_h, w1_h, b1_h, d0_h, m0_h, d1_h, m1_h, out_h,
                  w0_v, b0_v, w1_v, b1_v, scl_v, row_a, row_b,
                  sw_v, cw_v, s0_v, c0_v, rep_a, rep_b, emb1,
                  sem_in, sem):
        wid = lax.axis_index("s") * 2 + lax.axis_index("c")
        row0 = wid * ROWS_PER_W

        cps = [
            pltpu.async_copy(w0_h, w0_v, sem_in),
            pltpu.async_copy(b0_h, b0_v, sem_in),
            pltpu.async_copy(w1_h, w1_v, sem_in),
            pltpu.async_copy(b1_h, b1_v, sem_in),
            pltpu.async_copy(d0_h, scl_v.at[pl.ds(0, 1)], sem_in),
            pltpu.async_copy(m0_h, scl_v.at[pl.ds(8, 1)], sem_in),
            pltpu.async_copy(d1_h, scl_v.at[pl.ds(16, 1)], sem_in),
            pltpu.async_copy(m1_h, scl_v.at[pl.ds(24, 1)], sem_in),
        ]
        for cp in cps:
            cp.wait()

        scl0 = scl_v[pl.ds(0, LANES)]
        scl1 = scl_v[pl.ds(LANES, LANES)]
        scale0 = _lane_broadcast(scl0, 8) / _lane_broadcast(scl0, 0)
        scale1 = _lane_broadcast(scl1, 8) / _lane_broadcast(scl1, 0)

        pos_a = (2 * wid).astype(jnp.float32)

        STRIP = 4

        def fill_from_row(rep, row_buf):
            vs = [row_buf[pl.ds(j * LANES, LANES)] for j in range(NCHUNK)]

            def bc_body(r, _):
                for j in range(NCHUNK):
                    rep[r, pl.ds(j * LANES, LANES)] = vs[j]
                return 0

            lax.fori_loop(0, STRIP, bc_body, 0)

        def row_a_body(j, _):
            o = pl.multiple_of(j * LANES, LANES)
            w = w0_v[pl.ds(o, LANES)] * scale0
            b = b0_v[pl.ds(o, LANES)]
            row_a[pl.ds(o, LANES)] = _sin_vec(pos_a * w + b)
            return 0

        lax.fori_loop(0, NCHUNK, row_a_body, 0)

        fill_from_row(rep_a, row_a)
        dmas_a = [
            pltpu.async_copy(
                rep_a,
                out_h.at[pl.ds(row0 + STRIP * k, STRIP), pl.ds(0, HALF)], sem)
            for k in range(L1 // STRIP)
        ]

        def row_b_body(j, _):
            o = pl.multiple_of(j * LANES, LANES)
            w = w0_v[pl.ds(o, LANES)] * scale0
            b = b0_v[pl.ds(o, LANES)]
            row_b[pl.ds(o, LANES)] = _sin_vec((pos_a + 1.0) * w + b)
            return 0

        lax.fori_loop(0, NCHUNK, row_b_body, 0)
        fill_from_row(rep_b, row_b)
        dmas_b = [
            pltpu.async_copy(
                rep_b,
                out_h.at[pl.ds(row0 + L1 + STRIP * k, STRIP), pl.ds(0, HALF)],
                sem)
            for k in range(L1 // STRIP)
        ]

        def ang_body(j, _):
            o = pl.multiple_of(j * LANES, LANES)
            a = w1_v[pl.ds(o, LANES)] * scale1
            b = b1_v[pl.ds(o, LANES)]
            sw_v[pl.ds(o, LANES)] = _sin_vec(a)
            cw_v[pl.ds(o, LANES)] = _sin_vec(a + _HALF_PI)
            s0_v[pl.ds(o, LANES)] = _sin_vec(b)
            c0_v[pl.ds(o, LANES)] = _sin_vec(b + _HALF_PI)
            return 0

        lax.fori_loop(0, NCHUNK, ang_body, 0)

        GRP = 8

        def grp_body(jg, _):
            o = pl.multiple_of(jg * (GRP * LANES), GRP * LANES)
            ofs = [o + u * LANES for u in range(GRP)]
            sws = [sw_v[pl.ds(c, LANES)] for c in ofs]
            cws = [cw_v[pl.ds(c, LANES)] for c in ofs]
            ss = [s0_v[pl.ds(c, LANES)] for c in ofs]
            cs = [c0_v[pl.ds(c, LANES)] for c in ofs]
            for u in range(GRP):
                emb1[0, pl.ds(ofs[u], LANES)] = ss[u]

            def rec_body(r, carry):
                ss_c, cs_c = carry
                ns, nc = [], []
                for u in range(GRP):
                    s2 = ss_c[u] * cws[u] + cs_c[u] * sws[u]
                    c2 = cs_c[u] * cws[u] - ss_c[u] * sws[u]
                    emb1[r, pl.ds(ofs[u], LANES)] = s2
                    ns.append(s2)
                    nc.append(c2)
                return (tuple(ns), tuple(nc))

            lax.fori_loop(1, L1, rec_body, (tuple(ss), tuple(cs)))
            return 0

        lax.fori_loop(0, NCHUNK // GRP, grp_body, 0)

        dma_e1 = pltpu.async_copy(
            emb1, out_h.at[pl.ds(row0, L1), pl.ds(HALF, HALF)], sem)
        dma_e2 = pltpu.async_copy(
            emb1, out_h.at[pl.ds(row0 + L1, L1), pl.ds(HALF, HALF)], sem)

        for cp in dmas_a:
            cp.wait()
        for cp in dmas_b:
            cp.wait()
        dma_e1.wait()
        dma_e2.wait()

    return sc_kernel


_SC_KERNEL = _make_sc_kernel()


def kernel(seq_len_or_axial_dims, W0, b0, W1, b1, div0, mult0, div1, mult1):
    return _SC_KERNEL(
        jnp.reshape(W0, (HALF,)), b0,
        jnp.reshape(W1, (HALF,)), b1,
        jnp.reshape(div0, (1,)), jnp.reshape(mult0, (1,)),
        jnp.reshape(div1, (1,)), jnp.reshape(mult1, (1,)),
    )


# speedup vs baseline: 1.0138x; 1.0138x over previous
import functools

import jax
import jax.numpy as jnp
from jax import lax
from jax.experimental import pallas as pl
from jax.experimental.pallas import tpu as pltpu
from jax.experimental.pallas import tpu_sc as plsc

DIM = 1024
HALF = 512
L0 = 64
L1 = 64
TOTAL = L0 * L1
NW = 32
ROWS_PER_W = TOTAL // NW
LANES = 16
NCHUNK = HALF // LANES

_TWO_PI = 6.283185307179586
_PI = 3.141592653589793
_HALF_PI = 1.5707963267948966


def _sin_vec(x):
    y = x * (1.0 / _TWO_PI)
    k = jnp.where(y >= 0, y + 0.5, y - 0.5).astype(jnp.int32).astype(jnp.float32)
    r = x - k * _TWO_PI
    r = jnp.where(r > _HALF_PI, _PI - r, r)
    r = jnp.where(r < -_HALF_PI, -_PI - r, r)
    r2 = r * r
    p = jnp.float32(-2.5052108385441718e-08)
    p = p * r2 + 2.7557319223985893e-06
    p = p * r2 - 0.0001984126984126984
    p = p * r2 + 0.008333333333333333
    p = p * r2 - 0.16666666666666666
    p = p * r2 + 1.0
    return r * p


def _lane_broadcast(v, lane):
    idx = jnp.full((LANES, 1), lane, dtype=jnp.int32)
    dnums = lax.GatherDimensionNumbers(
        offset_dims=(), collapsed_slice_dims=(0,), start_index_map=(0,))
    return lax.gather(v, idx, dnums, (1,),
                      mode=lax.GatherScatterMode.PROMISE_IN_BOUNDS)


def _make_sc_kernel():
    mesh = plsc.VectorSubcoreMesh(core_axis_name="c", subcore_axis_name="s")

    @functools.partial(
        pl.kernel,
        mesh=mesh,
        out_type=jax.ShapeDtypeStruct((TOTAL, DIM), jnp.float32),
        scratch_types=[
            pltpu.VMEM((HALF,), jnp.float32),
            pltpu.VMEM((HALF,), jnp.float32),
            pltpu.VMEM((HALF,), jnp.float32),
            pltpu.VMEM((HALF,), jnp.float32),
            pltpu.VMEM((2 * LANES,), jnp.float32),
            pltpu.VMEM((HALF,), jnp.float32),
            pltpu.VMEM((HALF,), jnp.float32),
            pltpu.VMEM((HALF,), jnp.float32),
            pltpu.VMEM((HALF,), jnp.float32),
            pltpu.VMEM((HALF,), jnp.float32),
            pltpu.VMEM((HALF,), jnp.float32),
            pltpu.VMEM((8, HALF), jnp.float32),
            pltpu.VMEM((8, HALF), jnp.float32),
            pltpu.VMEM((L1, HALF), jnp.float32),
            pltpu.SemaphoreType.DMA,
            pltpu.SemaphoreType.DMA,
        ],
    )
    def sc_kernel(w0_h, b0_h, w1_h, b1_h, d0_h, m0_h, d1_h, m1_h, out_h,
                  w0_v, b0_v, w1_v, b1_v, scl_v, row_a, row_b,
                  sw_v, cw_v, s0_v, c0_v, rep_a, rep_b, emb1,
                  sem_in, sem):
        wid = lax.axis_index("s") * 2 + lax.axis_index("c")
        row0 = wid * ROWS_PER_W

        cps = [
            pltpu.async_copy(w0_h, w0_v, sem_in),
            pltpu.async_copy(b0_h, b0_v, sem_in),
            pltpu.async_copy(w1_h, w1_v, sem_in),
            pltpu.async_copy(b1_h, b1_v, sem_in),
            pltpu.async_copy(d0_h, scl_v.at[pl.ds(0, 1)], sem_in),
            pltpu.async_copy(m0_h, scl_v.at[pl.ds(8, 1)], sem_in),
            pltpu.async_copy(d1_h, scl_v.at[pl.ds(16, 1)], sem_in),
            pltpu.async_copy(m1_h, scl_v.at[pl.ds(24, 1)], sem_in),
        ]
        for cp in cps:
            cp.wait()

        scl0 = scl_v[pl.ds(0, LANES)]
        scl1 = scl_v[pl.ds(LANES, LANES)]
        scale0 = _lane_broadcast(scl0, 8) / _lane_broadcast(scl0, 0)
        scale1 = _lane_broadcast(scl1, 8) / _lane_broadcast(scl1, 0)

        pos_a = (2 * wid).astype(jnp.float32)

        STRIP = 8

        def fill_from_row(rep, row_buf):
            vs = [row_buf[pl.ds(j * LANES, LANES)] for j in range(NCHUNK)]

            def bc_body(r, _):
                for j in range(NCHUNK):
                    rep[r, pl.ds(j * LANES, LANES)] = vs[j]
                return 0

            lax.fori_loop(0, STRIP, bc_body, 0)

        def row_a_body(j, _):
            o = pl.multiple_of(j * LANES, LANES)
            w = w0_v[pl.ds(o, LANES)] * scale0
            b = b0_v[pl.ds(o, LANES)]
            row_a[pl.ds(o, LANES)] = _sin_vec(pos_a * w + b)
            return 0

        lax.fori_loop(0, NCHUNK, row_a_body, 0)

        fill_from_row(rep_a, row_a)
        dmas_a = [
            pltpu.async_copy(
                rep_a,
                out_h.at[pl.ds(row0 + STRIP * k, STRIP), pl.ds(0, HALF)], sem)
            for k in range(L1 // STRIP)
        ]

        def row_b_body(j, _):
            o = pl.multiple_of(j * LANES, LANES)
            w = w0_v[pl.ds(o, LANES)] * scale0
            b = b0_v[pl.ds(o, LANES)]
            row_b[pl.ds(o, LANES)] = _sin_vec((pos_a + 1.0) * w + b)
            return 0

        lax.fori_loop(0, NCHUNK, row_b_body, 0)
        fill_from_row(rep_b, row_b)
        dmas_b = [
            pltpu.async_copy(
                rep_b,
                out_h.at[pl.ds(row0 + L1 + STRIP * k, STRIP), pl.ds(0, HALF)],
                sem)
            for k in range(L1 // STRIP)
        ]

        def ang_body(j, _):
            o = pl.multiple_of(j * LANES, LANES)
            a = w1_v[pl.ds(o, LANES)] * scale1
            b = b1_v[pl.ds(o, LANES)]
            sw_v[pl.ds(o, LANES)] = _sin_vec(a)
            cw_v[pl.ds(o, LANES)] = _sin_vec(a + _HALF_PI)
            s0_v[pl.ds(o, LANES)] = _sin_vec(b)
            c0_v[pl.ds(o, LANES)] = _sin_vec(b + _HALF_PI)
            return 0

        lax.fori_loop(0, NCHUNK, ang_body, 0)

        GRP = 8

        def grp_body(jg, _):
            o = pl.multiple_of(jg * (GRP * LANES), GRP * LANES)
            ofs = [o + u * LANES for u in range(GRP)]
            sws = [sw_v[pl.ds(c, LANES)] for c in ofs]
            cws = [cw_v[pl.ds(c, LANES)] for c in ofs]
            ss = [s0_v[pl.ds(c, LANES)] for c in ofs]
            cs = [c0_v[pl.ds(c, LANES)] for c in ofs]
            for u in range(GRP):
                emb1[0, pl.ds(ofs[u], LANES)] = ss[u]

            def rec_body(r, carry):
                ss_c, cs_c = carry
                ns, nc = [], []
                for u in range(GRP):
                    s2 = ss_c[u] * cws[u] + cs_c[u] * sws[u]
                    c2 = cs_c[u] * cws[u] - ss_c[u] * sws[u]
                    emb1[r, pl.ds(ofs[u], LANES)] = s2
                    ns.append(s2)
                    nc.append(c2)
                return (tuple(ns), tuple(nc))

            lax.fori_loop(1, L1, rec_body, (tuple(ss), tuple(cs)))
            return 0

        lax.fori_loop(0, NCHUNK // GRP, grp_body, 0)

        dma_e1 = pltpu.async_copy(
            emb1, out_h.at[pl.ds(row0, L1), pl.ds(HALF, HALF)], sem)
        dma_e2 = pltpu.async_copy(
            emb1, out_h.at[pl.ds(row0 + L1, L1), pl.ds(HALF, HALF)], sem)

        for cp in dmas_a:
            cp.wait()
        for cp in dmas_b:
            cp.wait()
        dma_e1.wait()
        dma_e2.wait()

    return sc_kernel


_SC_KERNEL = _make_sc_kernel()


def kernel(seq_len_or_axial_dims, W0, b0, W1, b1, div0, mult0, div1, mult1):
    return _SC_KERNEL(
        jnp.reshape(W0, (HALF,)), b0,
        jnp.reshape(W1, (HALF,)), b1,
        jnp.reshape(div0, (1,)), jnp.reshape(mult0, (1,)),
        jnp.reshape(div1, (1,)), jnp.reshape(mult1, (1,)),
    )


# speedup vs baseline: 1.0324x; 1.0183x over previous
import functools

import jax
import jax.numpy as jnp
from jax import lax
from jax.experimental import pallas as pl
from jax.experimental.pallas import tpu as pltpu
from jax.experimental.pallas import tpu_sc as plsc

DIM = 1024
HALF = 512
L0 = 64
L1 = 64
TOTAL = L0 * L1
NW = 32
ROWS_PER_W = TOTAL // NW
LANES = 16
NCHUNK = HALF // LANES

_TWO_PI = 6.283185307179586
_PI = 3.141592653589793
_HALF_PI = 1.5707963267948966


def _sin_vec(x):
    y = x * (1.0 / _TWO_PI)
    k = jnp.where(y >= 0, y + 0.5, y - 0.5).astype(jnp.int32).astype(jnp.float32)
    r = x - k * _TWO_PI
    r = jnp.where(r > _HALF_PI, _PI - r, r)
    r = jnp.where(r < -_HALF_PI, -_PI - r, r)
    r2 = r * r
    p = jnp.float32(-2.5052108385441718e-08)
    p = p * r2 + 2.7557319223985893e-06
    p = p * r2 - 0.0001984126984126984
    p = p * r2 + 0.008333333333333333
    p = p * r2 - 0.16666666666666666
    p = p * r2 + 1.0
    return r * p


def _lane_broadcast(v, lane):
    idx = jnp.full((LANES, 1), lane, dtype=jnp.int32)
    dnums = lax.GatherDimensionNumbers(
        offset_dims=(), collapsed_slice_dims=(0,), start_index_map=(0,))
    return lax.gather(v, idx, dnums, (1,),
                      mode=lax.GatherScatterMode.PROMISE_IN_BOUNDS)


def _make_sc_kernel():
    mesh = plsc.VectorSubcoreMesh(core_axis_name="c", subcore_axis_name="s")

    @functools.partial(
        pl.kernel,
        mesh=mesh,
        out_type=jax.ShapeDtypeStruct((TOTAL, DIM), jnp.float32),
        scratch_types=[
            pltpu.VMEM((HALF,), jnp.float32),
            pltpu.VMEM((HALF,), jnp.float32),
            pltpu.VMEM((HALF,), jnp.float32),
            pltpu.VMEM((HALF,), jnp.float32),
            pltpu.VMEM((2 * LANES,), jnp.float32),
            pltpu.VMEM((HALF,), jnp.float32),
            pltpu.VMEM((HALF,), jnp.float32),
            pltpu.VMEM((HALF,), jnp.float32),
            pltpu.VMEM((HALF,), jnp.float32),
            pltpu.VMEM((HALF,), jnp.float32),
            pltpu.VMEM((HALF,), jnp.float32),
            pltpu.VMEM((8, HALF), jnp.float32),
            pltpu.VMEM((8, HALF), jnp.float32),
            pltpu.VMEM((L1, HALF), jnp.float32),
            pltpu.SemaphoreType.DMA,
            pltpu.SemaphoreType.DMA,
        ],
    )
    def sc_kernel(w0_h, b0_h, w1_h, b1_h, d0_h, m0_h, d1_h, m1_h, out_h,
                  w0_v, b0_v, w1_v, b1_v, scl_v, row_a, row_b,
                  sw_v, cw_v, s0_v, c0_v, rep_a, rep_b, emb1,
                  sem_in, sem):
        wid = lax.axis_index("s") * 2 + lax.axis_index("c")
        row0 = wid * ROWS_PER_W

        cps = [
            pltpu.async_copy(w0_h, w0_v, sem_in),
            pltpu.async_copy(b0_h, b0_v, sem_in),
            pltpu.async_copy(w1_h, w1_v, sem_in),
            pltpu.async_copy(b1_h, b1_v, sem_in),
            pltpu.async_copy(d0_h, scl_v.at[pl.ds(0, 1)], sem_in),
            pltpu.async_copy(m0_h, scl_v.at[pl.ds(8, 1)], sem_in),
            pltpu.async_copy(d1_h, scl_v.at[pl.ds(16, 1)], sem_in),
            pltpu.async_copy(m1_h, scl_v.at[pl.ds(24, 1)], sem_in),
        ]
        for cp in cps:
            cp.wait()

        scl0 = scl_v[pl.ds(0, LANES)]
        scl1 = scl_v[pl.ds(LANES, LANES)]
        scale0 = _lane_broadcast(scl0, 8) / _lane_broadcast(scl0, 0)
        scale1 = _lane_broadcast(scl1, 8) / _lane_broadcast(scl1, 0)

        pos_a = (2 * wid).astype(jnp.float32)

        STRIP = 8

        def fill_from_row(rep, row_buf):
            vs = [row_buf[pl.ds(j * LANES, LANES)] for j in range(NCHUNK)]

            def bc_body(r, _):
                for j in range(NCHUNK):
                    rep[r, pl.ds(j * LANES, LANES)] = vs[j]
                return 0

            lax.fori_loop(0, STRIP, bc_body, 0)

        def row_body(j, _):
            o = pl.multiple_of(j * LANES, LANES)
            w = w0_v[pl.ds(o, LANES)] * scale0
            b = b0_v[pl.ds(o, LANES)]
            row_a[pl.ds(o, LANES)] = _sin_vec(pos_a * w + b)
            row_b[pl.ds(o, LANES)] = _sin_vec((pos_a + 1.0) * w + b)
            return 0

        lax.fori_loop(0, NCHUNK, row_body, 0)

        fill_from_row(rep_a, row_a)
        dmas_a = [
            pltpu.async_copy(
                rep_a,
                out_h.at[pl.ds(row0 + STRIP * k, STRIP), pl.ds(0, HALF)], sem)
            for k in range(L1 // STRIP)
        ]
        fill_from_row(rep_b, row_b)
        dmas_b = [
            pltpu.async_copy(
                rep_b,
                out_h.at[pl.ds(row0 + L1 + STRIP * k, STRIP), pl.ds(0, HALF)],
                sem)
            for k in range(L1 // STRIP)
        ]

        def ang_body(j, _):
            o = pl.multiple_of(j * LANES, LANES)
            a = w1_v[pl.ds(o, LANES)] * scale1
            b = b1_v[pl.ds(o, LANES)]
            sw_v[pl.ds(o, LANES)] = _sin_vec(a)
            cw_v[pl.ds(o, LANES)] = _sin_vec(a + _HALF_PI)
            s0_v[pl.ds(o, LANES)] = _sin_vec(b)
            c0_v[pl.ds(o, LANES)] = _sin_vec(b + _HALF_PI)
            return 0

        lax.fori_loop(0, NCHUNK, ang_body, 0)

        GRP = 8

        def grp_body(jg, _):
            o = pl.multiple_of(jg * (GRP * LANES), GRP * LANES)
            ofs = [o + u * LANES for u in range(GRP)]
            sws = [sw_v[pl.ds(c, LANES)] for c in ofs]
            cws = [cw_v[pl.ds(c, LANES)] for c in ofs]
            ss = [s0_v[pl.ds(c, LANES)] for c in ofs]
            cs = [c0_v[pl.ds(c, LANES)] for c in ofs]
            for u in range(GRP):
                emb1[0, pl.ds(ofs[u], LANES)] = ss[u]

            def rec_body(r, carry):
                ss_c, cs_c = carry
                ns, nc = [], []
                for u in range(GRP):
                    s2 = ss_c[u] * cws[u] + cs_c[u] * sws[u]
                    c2 = cs_c[u] * cws[u] - ss_c[u] * sws[u]
                    emb1[r, pl.ds(ofs[u], LANES)] = s2
                    ns.append(s2)
                    nc.append(c2)
                return (tuple(ns), tuple(nc))

            lax.fori_loop(1, L1, rec_body, (tuple(ss), tuple(cs)))
            return 0

        lax.fori_loop(0, NCHUNK // GRP, grp_body, 0)

        dma_e1 = pltpu.async_copy(
            emb1, out_h.at[pl.ds(row0, L1), pl.ds(HALF, HALF)], sem)
        dma_e2 = pltpu.async_copy(
            emb1, out_h.at[pl.ds(row0 + L1, L1), pl.ds(HALF, HALF)], sem)

        for cp in dmas_a:
            cp.wait()
        for cp in dmas_b:
            cp.wait()
        dma_e1.wait()
        dma_e2.wait()

    return sc_kernel


_SC_KERNEL = _make_sc_kernel()


def kernel(seq_len_or_axial_dims, W0, b0, W1, b1, div0, mult0, div1, mult1):
    return _SC_KERNEL(
        jnp.reshape(W0, (HALF,)), b0,
        jnp.reshape(W1, (HALF,)), b1,
        jnp.reshape(div0, (1,)), jnp.reshape(mult0, (1,)),
        jnp.reshape(div1, (1,)), jnp.reshape(mult1, (1,)),
    )
